# baseline (speedup 1.0000x reference)
import jax
import jax.numpy as jnp
from jax import lax
from jax.experimental import pallas as pl
from jax.experimental.pallas import tpu as pltpu
from jax.experimental.pallas import tpu_sc as plsc

K = 1000000
D = 16
NTILES = 32
CHUNK_ROWS = 1200
BLOCKS = CHUNK_ROWS // 16
NFULL = K // CHUNK_ROWS
TAIL_START = K - CHUNK_ROWS
TAIL_BLOCK0 = (NFULL * CHUNK_ROWS - TAIL_START) // 16
TAIL_TILE = NFULL % NTILES


def _iota16():
    return lax.iota(jnp.int32, 16)


def _lane_gather(v, idx16):
    dn = lax.GatherDimensionNumbers(
        offset_dims=(), collapsed_slice_dims=(0,), start_index_map=(0,))
    return lax.gather(v, idx16[:, None], dn, slice_sizes=(1,),
                      mode=lax.GatherScatterMode.PROMISE_IN_BOUNDS)


def _splat(v, lane):
    return _lane_gather(v, jnp.full((16,), lane, jnp.int32))


def _merge_topk(cur_v, cur_i, blk_v, blk_i):
    sk, si = plsc.sort_key_val(blk_v, blk_i)
    rv = lax.rev(sk, (0,))
    ri = lax.rev(si, (0,))
    take_a = cur_v <= rv
    mv = jnp.where(take_a, cur_v, rv)
    mi = jnp.where(take_a, cur_i, ri)
    nv, ni = plsc.sort_key_val(mv, mi)
    return nv, ni, _splat(nv, 7)


def _sc_body(mnv_hbm, smp_hbm, outv_hbm, outi_hbm, buf, sbuf, resv, resi):
    cid = lax.axis_index("c")
    sid = lax.axis_index("s")
    wid = sid * 2 + cid

    pltpu.sync_copy(smp_hbm, sbuf)
    s = sbuf[...]
    iota = _iota16()
    inf = jnp.float32(jnp.inf)
    diag = [(iota + d) & 15 for d in range(D)]
    s_d = [_lane_gather(s, diag[d]) for d in range(D)]

    def block_dist2(row0):
        acc = jnp.zeros((16,), jnp.float32)
        row_ids = row0 + iota
        for d in range(D):
            col = plsc.load_gather(buf, [row_ids, diag[d]])
            diff = col - s_d[d]
            acc = acc + diff * diff
        return acc

    def consider(acc, gidx, cur_v, cur_i, tv):
        hit = jnp.any(acc < tv)

        def do(acc, gidx, cur_v, cur_i, tv):
            return _merge_topk(cur_v, cur_i, acc, gidx)

        def skip(acc, gidx, cur_v, cur_i, tv):
            return cur_v, cur_i, tv

        return lax.cond(hit, do, skip, acc, gidx, cur_v, cur_i, tv)

    def make_blk_body(start):
        def blk_body(b, carry):
            cur_v, cur_i, tv = carry
            acc = block_dist2(b * 16)
            gidx = start + b * 16 + iota
            return consider(acc, gidx, cur_v, cur_i, tv)
        return blk_body

    def chunk_body(j, carry):
        ch = wid + j * NTILES
        start = pl.multiple_of(ch * CHUNK_ROWS, 8)
        pltpu.sync_copy(mnv_hbm.at[pl.ds(start, CHUNK_ROWS)], buf)
        return lax.fori_loop(0, BLOCKS, make_blk_body(start), carry)

    init = (jnp.full((16,), inf), jnp.zeros((16,), jnp.int32),
            jnp.full((16,), inf))
    nchunks = (NFULL - wid + NTILES - 1) // NTILES
    cur_v, cur_i, tv = lax.fori_loop(0, nchunks, chunk_body, init)

    @pl.when(wid == TAIL_TILE)
    def _():
        pltpu.sync_copy(mnv_hbm.at[pl.ds(TAIL_START, CHUNK_ROWS)], buf)
        cv, ci, _t = lax.fori_loop(
            TAIL_BLOCK0, BLOCKS, make_blk_body(TAIL_START), (cur_v, cur_i, tv))
        resv[...] = cv
        resi[...] = ci

    @pl.when(wid != TAIL_TILE)
    def _():
        resv[...] = cur_v
        resi[...] = cur_i

    pltpu.sync_copy(resv, outv_hbm.at[pl.ds(wid * 16, 16)])
    pltpu.sync_copy(resi, outi_hbm.at[pl.ds(wid * 16, 16)])


def _sc_dist_topk(mnv, smp):
    mesh = plsc.VectorSubcoreMesh(core_axis_name="c", subcore_axis_name="s")
    f = pl.kernel(
        _sc_body,
        mesh=mesh,
        compiler_params=pltpu.CompilerParams(
            needs_layout_passes=False, use_tc_tiling_on_sc=False),
        out_type=[
            jax.ShapeDtypeStruct((NTILES * 16,), jnp.float32),
            jax.ShapeDtypeStruct((NTILES * 16,), jnp.int32),
        ],
        scratch_types=[
            pltpu.VMEM((CHUNK_ROWS, D), jnp.float32),
            pltpu.VMEM((16,), jnp.float32),
            pltpu.VMEM((16,), jnp.float32),
            pltpu.VMEM((16,), jnp.int32),
        ],
    )
    return f(mnv, smp)


def _tc_merge_body(v_ref, i_ref, idx_ref, val_ref):
    V = v_ref[...]
    I = i_ref[...]
    big = jnp.int32(2**31 - 1)
    inf = jnp.float32(jnp.inf)
    idxs = []
    vals = []
    for _ in range(8):
        m = jnp.min(V)
        sel = V == m
        ci = jnp.min(jnp.where(sel, I, big))
        idxs.append(ci)
        vals.append(m)
        V = jnp.where(sel & (I == ci), inf, V)
    idx_ref[...] = jnp.stack(idxs)
    val_ref[...] = jnp.sqrt(jnp.stack(vals))


def _tc_merge(v2d, i2d):
    return pl.pallas_call(
        _tc_merge_body,
        out_shape=[
            jax.ShapeDtypeStruct((8,), jnp.int32),
            jax.ShapeDtypeStruct((8,), jnp.float32),
        ],
    )(v2d, i2d)


def kernel(samples, map_node_values, n):
    cv, ci = _sc_dist_topk(map_node_values, samples)
    idx, vals = _tc_merge(cv.reshape(NTILES, 16), ci.reshape(NTILES, 16))
    return idx, vals


# speedup vs baseline: 1.0033x; 1.0033x over previous
import jax
import jax.numpy as jnp
from jax import lax
from jax.experimental import pallas as pl
from jax.experimental.pallas import tpu as pltpu
from jax.experimental.pallas import tpu_sc as plsc

K = 1000000
D = 16
NTILES = 32
ROWS_PER_SUPER = 8
CHUNK_ROWS = 1280
CHUNK_SUPERS = CHUNK_ROWS // ROWS_PER_SUPER
BLOCKS = CHUNK_ROWS // 16
NFULL = K // CHUNK_ROWS
TAIL_ROW_START = K - CHUNK_ROWS
TAIL_SUPER_START = TAIL_ROW_START // ROWS_PER_SUPER
TAIL_BLOCK0 = (NFULL * CHUNK_ROWS - TAIL_ROW_START) // 16
TAIL_TILE = NFULL % NTILES


def _iota16():
    return lax.iota(jnp.int32, 16)


def _lane_gather(v, idx16):
    dn = lax.GatherDimensionNumbers(
        offset_dims=(), collapsed_slice_dims=(0,), start_index_map=(0,))
    return lax.gather(v, idx16[:, None], dn, slice_sizes=(1,),
                      mode=lax.GatherScatterMode.PROMISE_IN_BOUNDS)


def _splat(v, lane):
    return _lane_gather(v, jnp.full((16,), lane, jnp.int32))


def _merge_topk(cur_v, cur_i, blk_v, blk_i):
    sk, si = plsc.sort_key_val(blk_v, blk_i)
    rv = lax.rev(sk, (0,))
    ri = lax.rev(si, (0,))
    take_a = cur_v <= rv
    mv = jnp.where(take_a, cur_v, rv)
    mi = jnp.where(take_a, cur_i, ri)
    nv, ni = plsc.sort_key_val(mv, mi)
    return nv, ni, _splat(nv, 7)


def _sc_body(mnv_hbm, smp_hbm, outv_hbm, outi_hbm, buf, sbuf, resv, resi):
    cid = lax.axis_index("c")
    sid = lax.axis_index("s")
    wid = sid * 2 + cid

    pltpu.sync_copy(smp_hbm, sbuf)
    s = sbuf[...]
    iota = _iota16()
    inf = jnp.float32(jnp.inf)
    diag = [(iota + d) & 15 for d in range(D)]
    s_d = [_lane_gather(s, diag[d]) for d in range(D)]
    sup8 = iota >> 3
    col_d = [(iota * 16 + diag[d]) & 127 for d in range(D)]

    def block_dist2(b):
        sup_ids = b * 2 + sup8
        acc = jnp.zeros((16,), jnp.float32)
        for d in range(D):
            col = plsc.load_gather(buf, [sup_ids, col_d[d]])
            diff = col - s_d[d]
            acc = acc + diff * diff
        return acc

    def consider(acc, gidx, cur_v, cur_i, tv):
        hit = jnp.any(acc < tv)

        def do(acc, gidx, cur_v, cur_i, tv):
            return _merge_topk(cur_v, cur_i, acc, gidx)

        def skip(acc, gidx, cur_v, cur_i, tv):
            return cur_v, cur_i, tv

        return lax.cond(hit, do, skip, acc, gidx, cur_v, cur_i, tv)

    def make_blk_body(row_start):
        def blk_body(b, carry):
            cur_v, cur_i, tv = carry
            acc = block_dist2(b)
            gidx = row_start + b * 16 + iota
            return consider(acc, gidx, cur_v, cur_i, tv)
        return blk_body

    def chunk_body(j, carry):
        ch = wid + j * NTILES
        sstart = ch * CHUNK_SUPERS
        pltpu.sync_copy(mnv_hbm.at[pl.ds(sstart, CHUNK_SUPERS)], buf)
        return lax.fori_loop(
            0, BLOCKS, make_blk_body(ch * CHUNK_ROWS), carry)

    init = (jnp.full((16,), inf), jnp.zeros((16,), jnp.int32),
            jnp.full((16,), inf))
    nchunks = (NFULL - wid + NTILES - 1) // NTILES
    cur_v, cur_i, tv = lax.fori_loop(0, nchunks, chunk_body, init)

    @pl.when(wid == TAIL_TILE)
    def _():
        pltpu.sync_copy(
            mnv_hbm.at[pl.ds(TAIL_SUPER_START, CHUNK_SUPERS)], buf)
        cv, ci, _t = lax.fori_loop(
            TAIL_BLOCK0, BLOCKS, make_blk_body(TAIL_ROW_START),
            (cur_v, cur_i, tv))
        resv[...] = cv
        resi[...] = ci

    @pl.when(wid != TAIL_TILE)
    def _():
        resv[...] = cur_v
        resi[...] = cur_i

    pltpu.sync_copy(resv, outv_hbm.at[pl.ds(wid * 16, 16)])
    pltpu.sync_copy(resi, outi_hbm.at[pl.ds(wid * 16, 16)])


def _sc_dist_topk(mnv128, smp):
    mesh = plsc.VectorSubcoreMesh(core_axis_name="c", subcore_axis_name="s")
    f = pl.kernel(
        _sc_body,
        mesh=mesh,
        compiler_params=pltpu.CompilerParams(
            needs_layout_passes=False, use_tc_tiling_on_sc=True),
        out_type=[
            jax.ShapeDtypeStruct((NTILES * 16,), jnp.float32),
            jax.ShapeDtypeStruct((NTILES * 16,), jnp.int32),
        ],
        scratch_types=[
            pltpu.VMEM((CHUNK_SUPERS, 128), jnp.float32),
            pltpu.VMEM((16,), jnp.float32),
            pltpu.VMEM((16,), jnp.float32),
            pltpu.VMEM((16,), jnp.int32),
        ],
    )
    return f(mnv128, smp)


def _tc_merge_body(v_ref, i_ref, idx_ref, val_ref):
    V = v_ref[...]
    I = i_ref[...]
    big = jnp.int32(2**31 - 1)
    inf = jnp.float32(jnp.inf)
    idxs = []
    vals = []
    for _ in range(8):
        m = jnp.min(V)
        sel = V == m
        ci = jnp.min(jnp.where(sel, I, big))
        idxs.append(ci)
        vals.append(m)
        V = jnp.where(sel & (I == ci), inf, V)
    idx_ref[...] = jnp.stack(idxs)
    val_ref[...] = jnp.sqrt(jnp.stack(vals))


def _tc_merge(v2d, i2d):
    return pl.pallas_call(
        _tc_merge_body,
        out_shape=[
            jax.ShapeDtypeStruct((8,), jnp.int32),
            jax.ShapeDtypeStruct((8,), jnp.float32),
        ],
    )(v2d, i2d)


def kernel(samples, map_node_values, n):
    mnv128 = map_node_values.reshape(K // ROWS_PER_SUPER, 128)
    cv, ci = _sc_dist_topk(mnv128, samples)
    idx, vals = _tc_merge(cv.reshape(NTILES, 16), ci.reshape(NTILES, 16))
    return idx, vals


# speedup vs baseline: 4.1471x; 4.1336x over previous
import jax
import jax.numpy as jnp
from jax import lax
from jax.experimental import pallas as pl
from jax.experimental.pallas import tpu as pltpu
from jax.experimental.pallas import tpu_sc as plsc

K = 1000000
D = 16
NTILES = 32
CHUNK = 2048
SC_ROWS = 999936
NFULL = SC_ROWS // CHUNK
HALF_START = NFULL * CHUNK
HALF_ROWS = SC_ROWS - HALF_START
HALF_TILE = NFULL % NTILES
TC_START = SC_ROWS


def _iota16():
    return lax.iota(jnp.int32, 16)


def _lane_gather(v, idx16):
    dn = lax.GatherDimensionNumbers(
        offset_dims=(), collapsed_slice_dims=(0,), start_index_map=(0,))
    return lax.gather(v, idx16[:, None], dn, slice_sizes=(1,),
                      mode=lax.GatherScatterMode.PROMISE_IN_BOUNDS)


def _splat(v, lane):
    return _lane_gather(v, jnp.full((16,), lane, jnp.int32))


def _merge_topk(cur_v, cur_i, blk_v, blk_i):
    sk, si = plsc.sort_key_val(blk_v, blk_i)
    rv = lax.rev(sk, (0,))
    ri = lax.rev(si, (0,))
    take_a = cur_v <= rv
    mv = jnp.where(take_a, cur_v, rv)
    mi = jnp.where(take_a, cur_i, ri)
    nv, ni = plsc.sort_key_val(mv, mi)
    return nv, ni, _splat(nv, 7)


def _sc_body(mnvT_hbm, smp_hbm, outv_hbm, outi_hbm, buf, sbuf, resv, resi):
    cid = lax.axis_index("c")
    sid = lax.axis_index("s")
    wid = sid * 2 + cid

    pltpu.sync_copy(smp_hbm, sbuf)
    s = sbuf[...]
    iota = _iota16()
    inf = jnp.float32(jnp.inf)
    s_d = [_splat(s, d) for d in range(D)]

    def block_dist2(b):
        lr0 = b * 16
        acc = jnp.zeros((16,), jnp.float32)
        for d in range(D):
            col = buf[d, pl.ds(lr0, 16)]
            diff = col - s_d[d]
            acc = acc + diff * diff
        return acc

    def consider(acc, gidx, cur_v, cur_i, tv):
        hit = jnp.any(acc < tv)

        def do(acc, gidx, cur_v, cur_i, tv):
            return _merge_topk(cur_v, cur_i, acc, gidx)

        def skip(acc, gidx, cur_v, cur_i, tv):
            return cur_v, cur_i, tv

        return lax.cond(hit, do, skip, acc, gidx, cur_v, cur_i, tv)

    def make_blk_body(row_start):
        def blk_body(b, carry):
            cur_v, cur_i, tv = carry
            acc = block_dist2(b)
            gidx = row_start + b * 16 + iota
            return consider(acc, gidx, cur_v, cur_i, tv)
        return blk_body

    def chunk_body(j, carry):
        ch = wid + j * NTILES
        cstart = ch * CHUNK
        pltpu.sync_copy(mnvT_hbm.at[:, pl.ds(cstart, CHUNK)], buf)
        return lax.fori_loop(
            0, CHUNK // 16, make_blk_body(cstart), carry)

    init = (jnp.full((16,), inf), jnp.zeros((16,), jnp.int32),
            jnp.full((16,), inf))
    nchunks = (NFULL - wid + NTILES - 1) // NTILES
    cur_v, cur_i, tv = lax.fori_loop(0, nchunks, chunk_body, init)

    @pl.when(wid == HALF_TILE)
    def _():
        pltpu.sync_copy(
            mnvT_hbm.at[:, pl.ds(HALF_START, HALF_ROWS)],
            buf.at[:, pl.ds(0, HALF_ROWS)])
        cv, ci, _t = lax.fori_loop(
            0, HALF_ROWS // 16, make_blk_body(HALF_START),
            (cur_v, cur_i, tv))
        resv[...] = cv
        resi[...] = ci

    @pl.when(wid != HALF_TILE)
    def _():
        resv[...] = cur_v
        resi[...] = cur_i

    pltpu.sync_copy(resv, outv_hbm.at[pl.ds(wid * 16, 16)])
    pltpu.sync_copy(resi, outi_hbm.at[pl.ds(wid * 16, 16)])


def _sc_dist_topk(mnvT, smp):
    mesh = plsc.VectorSubcoreMesh(core_axis_name="c", subcore_axis_name="s")
    f = pl.kernel(
        _sc_body,
        mesh=mesh,
        compiler_params=pltpu.CompilerParams(
            needs_layout_passes=False, use_tc_tiling_on_sc=True),
        out_type=[
            jax.ShapeDtypeStruct((NTILES * 16,), jnp.float32),
            jax.ShapeDtypeStruct((NTILES * 16,), jnp.int32),
        ],
        scratch_types=[
            pltpu.VMEM((D, CHUNK), jnp.float32),
            pltpu.VMEM((16,), jnp.float32),
            pltpu.VMEM((16,), jnp.float32),
            pltpu.VMEM((16,), jnp.int32),
        ],
    )
    return f(mnvT, smp)


def _tc_merge_body(v_ref, i_ref, tail_ref, s_ref, idx_ref, val_ref):
    t = tail_ref[...]
    sv = s_ref[...]
    td = jnp.sum((t - sv) ** 2, axis=1)
    td = td.reshape(4, 16)
    ti = TC_START + lax.broadcasted_iota(jnp.int32, (4, 16), 0) * 16 \
        + lax.broadcasted_iota(jnp.int32, (4, 16), 1)
    V = jnp.concatenate([v_ref[...], td], axis=0)
    I = jnp.concatenate([i_ref[...], ti], axis=0)
    big = jnp.int32(2**31 - 1)
    inf = jnp.float32(jnp.inf)
    idxs = []
    vals = []
    for _ in range(8):
        m = jnp.min(V)
        sel = V == m
        ci = jnp.min(jnp.where(sel, I, big))
        idxs.append(ci)
        vals.append(m)
        V = jnp.where(sel & (I == ci), inf, V)
    idx_ref[...] = jnp.stack(idxs)
    val_ref[...] = jnp.sqrt(jnp.stack(vals))


def _tc_merge(v2d, i2d, tail, smp):
    return pl.pallas_call(
        _tc_merge_body,
        out_shape=[
            jax.ShapeDtypeStruct((8,), jnp.int32),
            jax.ShapeDtypeStruct((8,), jnp.float32),
        ],
    )(v2d, i2d, tail, smp)


def kernel(samples, map_node_values, n):
    mnvT = map_node_values.T
    cv, ci = _sc_dist_topk(mnvT, samples)
    tail = map_node_values[TC_START:]
    idx, vals = _tc_merge(cv.reshape(NTILES, 16), ci.reshape(NTILES, 16),
                          tail, samples.reshape(1, D))
    return idx, vals


# speedup vs baseline: 5.0765x; 1.2241x over previous
import jax
import jax.numpy as jnp
from jax import lax
from jax.experimental import pallas as pl
from jax.experimental.pallas import tpu as pltpu
from jax.experimental.pallas import tpu_sc as plsc

K = 1000000
D = 16
NTILES = 32
CHUNK = 2048
SC_ROWS = 999936
NFULL = SC_ROWS // CHUNK
HALF_START = NFULL * CHUNK
HALF_ROWS = SC_ROWS - HALF_START
HALF_TILE = NFULL % NTILES
TC_START = SC_ROWS


def _iota16():
    return lax.iota(jnp.int32, 16)


def _lane_gather(v, idx16):
    dn = lax.GatherDimensionNumbers(
        offset_dims=(), collapsed_slice_dims=(0,), start_index_map=(0,))
    return lax.gather(v, idx16[:, None], dn, slice_sizes=(1,),
                      mode=lax.GatherScatterMode.PROMISE_IN_BOUNDS)


def _splat(v, lane):
    return _lane_gather(v, jnp.full((16,), lane, jnp.int32))


def _merge_topk(cur_v, cur_i, blk_v, blk_i):
    sk, si = plsc.sort_key_val(blk_v, blk_i)
    rv = lax.rev(sk, (0,))
    ri = lax.rev(si, (0,))
    take_a = cur_v <= rv
    mv = jnp.where(take_a, cur_v, rv)
    mi = jnp.where(take_a, cur_i, ri)
    nv, ni = plsc.sort_key_val(mv, mi)
    return nv, ni, _splat(nv, 7)


def _sc_body(mnvT_hbm, smp_hbm, outv_hbm, outi_hbm,
             bufa, bufb, sbuf, resv, resi, sema, semb):
    cid = lax.axis_index("c")
    sid = lax.axis_index("s")
    wid = sid * 2 + cid

    pltpu.sync_copy(smp_hbm, sbuf)
    s = sbuf[...]
    iota = _iota16()
    inf = jnp.float32(jnp.inf)
    s_d = [_splat(s, d) for d in range(D)]

    def block_dist2(buf, b):
        lr0 = b * 16
        acc = jnp.zeros((16,), jnp.float32)
        for d in range(D):
            col = buf[d, pl.ds(lr0, 16)]
            diff = col - s_d[d]
            acc = acc + diff * diff
        return acc

    def consider(ok, acc, gidx, cur_v, cur_i, tv):
        hit = ok & jnp.any(acc < tv)

        def do(acc, gidx, cur_v, cur_i, tv):
            return _merge_topk(cur_v, cur_i, acc, gidx)

        def skip(acc, gidx, cur_v, cur_i, tv):
            return cur_v, cur_i, tv

        return lax.cond(hit, do, skip, acc, gidx, cur_v, cur_i, tv)

    def make_blk_body(buf, ok, row_start):
        def blk_body(b, carry):
            cur_v, cur_i, tv = carry
            acc = block_dist2(buf, b)
            gidx = row_start + b * 16 + iota
            return consider(ok, acc, gidx, cur_v, cur_i, tv)
        return blk_body

    def slot_chunk(j):
        ch = wid + j * NTILES
        ok = ch < NFULL
        return jnp.where(ok, ch, wid), ok

    def dma(ch, buf, sem):
        return pltpu.make_async_copy(
            mnvT_hbm.at[:, pl.ds(ch * CHUNK, CHUNK)], buf, sem)

    def compute(ch, ok, buf, carry):
        return lax.fori_loop(
            0, CHUNK // 16, make_blk_body(buf, ok, ch * CHUNK), carry)

    NSLOT = (NFULL + NTILES - 1) // NTILES

    ch0, _ = slot_chunk(0)
    dma(ch0, bufa, sema).start()

    def pair_body(p, carry):
        j0 = p * 2
        ch0, ok0 = slot_chunk(j0)
        ch1, ok1 = slot_chunk(j0 + 1)
        ch2, _ = slot_chunk(j0 + 2)
        dma(ch0, bufa, sema).wait()
        dma(ch1, bufb, semb).start()
        carry = compute(ch0, ok0, bufa, carry)
        dma(ch1, bufb, semb).wait()
        dma(ch2, bufa, sema).start()
        return compute(ch1, ok1, bufb, carry)

    init = (jnp.full((16,), inf), jnp.zeros((16,), jnp.int32),
            jnp.full((16,), inf))
    cur_v, cur_i, tv = lax.fori_loop(0, NSLOT // 2, pair_body, init)
    chx, _ = slot_chunk(NSLOT)
    dma(chx, bufa, sema).wait()

    @pl.when(wid == HALF_TILE)
    def _():
        pltpu.sync_copy(
            mnvT_hbm.at[:, pl.ds(HALF_START, HALF_ROWS)],
            bufa.at[:, pl.ds(0, HALF_ROWS)])
        cv, ci, _t = lax.fori_loop(
            0, HALF_ROWS // 16,
            make_blk_body(bufa, True, HALF_START), (cur_v, cur_i, tv))
        resv[...] = cv
        resi[...] = ci

    @pl.when(wid != HALF_TILE)
    def _():
        resv[...] = cur_v
        resi[...] = cur_i

    pltpu.sync_copy(resv, outv_hbm.at[pl.ds(wid * 16, 16)])
    pltpu.sync_copy(resi, outi_hbm.at[pl.ds(wid * 16, 16)])


def _sc_dist_topk(mnvT, smp):
    mesh = plsc.VectorSubcoreMesh(core_axis_name="c", subcore_axis_name="s")
    f = pl.kernel(
        _sc_body,
        mesh=mesh,
        compiler_params=pltpu.CompilerParams(
            needs_layout_passes=False, use_tc_tiling_on_sc=True),
        out_type=[
            jax.ShapeDtypeStruct((NTILES * 16,), jnp.float32),
            jax.ShapeDtypeStruct((NTILES * 16,), jnp.int32),
        ],
        scratch_types=[
            pltpu.VMEM((D, CHUNK), jnp.float32),
            pltpu.VMEM((D, CHUNK), jnp.float32),
            pltpu.VMEM((16,), jnp.float32),
            pltpu.VMEM((16,), jnp.float32),
            pltpu.VMEM((16,), jnp.int32),
            pltpu.SemaphoreType.DMA,
            pltpu.SemaphoreType.DMA,
        ],
    )
    return f(mnvT, smp)


def _tc_merge_body(v_ref, i_ref, tail_ref, s_ref, idx_ref, val_ref):
    t = tail_ref[...]
    sv = s_ref[...]
    td = jnp.sum((t - sv) ** 2, axis=1)
    td = td.reshape(4, 16)
    ti = TC_START + lax.broadcasted_iota(jnp.int32, (4, 16), 0) * 16 \
        + lax.broadcasted_iota(jnp.int32, (4, 16), 1)
    V = jnp.concatenate([v_ref[...], td], axis=0)
    I = jnp.concatenate([i_ref[...], ti], axis=0)
    big = jnp.int32(2**31 - 1)
    inf = jnp.float32(jnp.inf)
    idxs = []
    vals = []
    for _ in range(8):
        m = jnp.min(V)
        sel = V == m
        ci = jnp.min(jnp.where(sel, I, big))
        idxs.append(ci)
        vals.append(m)
        V = jnp.where(sel & (I == ci), inf, V)
    idx_ref[...] = jnp.stack(idxs)
    val_ref[...] = jnp.sqrt(jnp.stack(vals))


def _tc_merge(v2d, i2d, tail, smp):
    return pl.pallas_call(
        _tc_merge_body,
        out_shape=[
            jax.ShapeDtypeStruct((8,), jnp.int32),
            jax.ShapeDtypeStruct((8,), jnp.float32),
        ],
    )(v2d, i2d, tail, smp)


def kernel(samples, map_node_values, n):
    mnvT = map_node_values.T
    cv, ci = _sc_dist_topk(mnvT, samples)
    tail = map_node_values[TC_START:]
    idx, vals = _tc_merge(cv.reshape(NTILES, 16), ci.reshape(NTILES, 16),
                          tail, samples.reshape(1, D))
    return idx, vals


# speedup vs baseline: 6.9560x; 1.3702x over previous
import jax
import jax.numpy as jnp
from jax import lax
from jax.experimental import pallas as pl
from jax.experimental.pallas import tpu as pltpu
from jax.experimental.pallas import tpu_sc as plsc

K = 1000000
D = 16
NTILES = 32
CHUNK = 2048
SC_ROWS = 999936
NFULL = SC_ROWS // CHUNK
HALF_START = NFULL * CHUNK
HALF_ROWS = SC_ROWS - HALF_START
HALF_TILE = NFULL % NTILES
TC_START = SC_ROWS


def _iota16():
    return lax.iota(jnp.int32, 16)


def _lane_gather(v, idx16):
    dn = lax.GatherDimensionNumbers(
        offset_dims=(), collapsed_slice_dims=(0,), start_index_map=(0,))
    return lax.gather(v, idx16[:, None], dn, slice_sizes=(1,),
                      mode=lax.GatherScatterMode.PROMISE_IN_BOUNDS)


def _splat(v, lane):
    return _lane_gather(v, jnp.full((16,), lane, jnp.int32))


def _merge_topk(cur_v, cur_i, blk_v, blk_i):
    sk, si = plsc.sort_key_val(blk_v, blk_i)
    rv = lax.rev(sk, (0,))
    ri = lax.rev(si, (0,))
    take_a = cur_v <= rv
    mv = jnp.where(take_a, cur_v, rv)
    mi = jnp.where(take_a, cur_i, ri)
    nv, ni = plsc.sort_key_val(mv, mi)
    return nv, ni, _splat(nv, 7)


def _sc_body(mnvT_hbm, smp_hbm, outv_hbm, outi_hbm,
             bufa, bufb, sbuf, resv, resi, sema, semb):
    cid = lax.axis_index("c")
    sid = lax.axis_index("s")
    wid = sid * 2 + cid

    pltpu.sync_copy(smp_hbm, sbuf)
    s = sbuf[...]
    iota = _iota16()
    inf = jnp.float32(jnp.inf)
    s_d = [_splat(s, d) for d in range(D)]

    def block_dist2(buf, b):
        lr0 = b * 16
        accs = [jnp.zeros((16,), jnp.float32) for _ in range(4)]
        for d in range(D):
            col = buf[d, pl.ds(lr0, 16)]
            diff = col - s_d[d]
            accs[d & 3] = accs[d & 3] + diff * diff
        return (accs[0] + accs[1]) + (accs[2] + accs[3])

    def make_blk_body(buf, ok, row_start):
        def blk_body(b2, carry):
            cur_v, cur_i, tv = carry
            b = b2 * 2
            acc0 = block_dist2(buf, b)
            acc1 = block_dist2(buf, b + 1)
            gidx0 = row_start + b * 16 + iota
            hit = ok & jnp.any(jnp.minimum(acc0, acc1) < tv)

            def do(acc0, acc1, gidx0, cur_v, cur_i, tv):
                cur_v, cur_i, tv = _merge_topk(cur_v, cur_i, acc0, gidx0)
                return _merge_topk(cur_v, cur_i, acc1, gidx0 + 16)

            def skip(acc0, acc1, gidx0, cur_v, cur_i, tv):
                return cur_v, cur_i, tv

            return lax.cond(hit, do, skip,
                            acc0, acc1, gidx0, cur_v, cur_i, tv)
        return blk_body

    def slot_chunk(j):
        ch = wid + j * NTILES
        ok = ch < NFULL
        return jnp.where(ok, ch, wid), ok

    def dma(ch, buf, sem):
        return pltpu.make_async_copy(
            mnvT_hbm.at[:, pl.ds(ch * CHUNK, CHUNK)], buf, sem)

    def compute(ch, ok, buf, carry):
        return lax.fori_loop(
            0, CHUNK // 32, make_blk_body(buf, ok, ch * CHUNK), carry)

    NSLOT = (NFULL + NTILES - 1) // NTILES

    ch0, _ = slot_chunk(0)
    dma(ch0, bufa, sema).start()

    def pair_body(p, carry):
        j0 = p * 2
        ch0, ok0 = slot_chunk(j0)
        ch1, ok1 = slot_chunk(j0 + 1)
        ch2, _ = slot_chunk(j0 + 2)
        dma(ch0, bufa, sema).wait()
        dma(ch1, bufb, semb).start()
        carry = compute(ch0, ok0, bufa, carry)
        dma(ch1, bufb, semb).wait()
        dma(ch2, bufa, sema).start()
        return compute(ch1, ok1, bufb, carry)

    init = (jnp.full((16,), inf), jnp.zeros((16,), jnp.int32),
            jnp.full((16,), inf))
    cur_v, cur_i, tv = lax.fori_loop(0, NSLOT // 2, pair_body, init)
    chx, _ = slot_chunk(NSLOT)
    dma(chx, bufa, sema).wait()

    @pl.when(wid == HALF_TILE)
    def _():
        pltpu.sync_copy(
            mnvT_hbm.at[:, pl.ds(HALF_START, HALF_ROWS)],
            bufa.at[:, pl.ds(0, HALF_ROWS)])
        cv, ci, _t = lax.fori_loop(
            0, HALF_ROWS // 32,
            make_blk_body(bufa, True, HALF_START), (cur_v, cur_i, tv))
        resv[...] = cv
        resi[...] = ci

    @pl.when(wid != HALF_TILE)
    def _():
        resv[...] = cur_v
        resi[...] = cur_i

    pltpu.sync_copy(resv, outv_hbm.at[pl.ds(wid * 16, 16)])
    pltpu.sync_copy(resi, outi_hbm.at[pl.ds(wid * 16, 16)])


def _sc_dist_topk(mnvT, smp):
    mesh = plsc.VectorSubcoreMesh(core_axis_name="c", subcore_axis_name="s")
    f = pl.kernel(
        _sc_body,
        mesh=mesh,
        compiler_params=pltpu.CompilerParams(
            needs_layout_passes=False, use_tc_tiling_on_sc=True),
        out_type=[
            jax.ShapeDtypeStruct((NTILES * 16,), jnp.float32),
            jax.ShapeDtypeStruct((NTILES * 16,), jnp.int32),
        ],
        scratch_types=[
            pltpu.VMEM((D, CHUNK), jnp.float32),
            pltpu.VMEM((D, CHUNK), jnp.float32),
            pltpu.VMEM((16,), jnp.float32),
            pltpu.VMEM((16,), jnp.float32),
            pltpu.VMEM((16,), jnp.int32),
            pltpu.SemaphoreType.DMA,
            pltpu.SemaphoreType.DMA,
        ],
    )
    return f(mnvT, smp)


def _tc_merge_body(v_ref, i_ref, tail_ref, s_ref, idx_ref, val_ref):
    t = tail_ref[...]
    sv = s_ref[...]
    td = jnp.sum((t - sv) ** 2, axis=1)
    td = td.reshape(4, 16)
    ti = TC_START + lax.broadcasted_iota(jnp.int32, (4, 16), 0) * 16 \
        + lax.broadcasted_iota(jnp.int32, (4, 16), 1)
    V = jnp.concatenate([v_ref[...], td], axis=0)
    I = jnp.concatenate([i_ref[...], ti], axis=0)
    big = jnp.int32(2**31 - 1)
    inf = jnp.float32(jnp.inf)
    idxs = []
    vals = []
    for _ in range(8):
        m = jnp.min(V)
        sel = V == m
        ci = jnp.min(jnp.where(sel, I, big))
        idxs.append(ci)
        vals.append(m)
        V = jnp.where(sel & (I == ci), inf, V)
    idx_ref[...] = jnp.stack(idxs)
    val_ref[...] = jnp.sqrt(jnp.stack(vals))


def _tc_merge(v2d, i2d, tail, smp):
    return pl.pallas_call(
        _tc_merge_body,
        out_shape=[
            jax.ShapeDtypeStruct((8,), jnp.int32),
            jax.ShapeDtypeStruct((8,), jnp.float32),
        ],
    )(v2d, i2d, tail, smp)


def kernel(samples, map_node_values, n):
    mnvT = map_node_values.T
    cv, ci = _sc_dist_topk(mnvT, samples)
    tail = map_node_values[TC_START:]
    idx, vals = _tc_merge(cv.reshape(NTILES, 16), ci.reshape(NTILES, 16),
                          tail, samples.reshape(1, D))
    return idx, vals


# speedup vs baseline: 8.3802x; 1.2048x over previous
import jax
import jax.numpy as jnp
from jax import lax
from jax.experimental import pallas as pl
from jax.experimental.pallas import tpu as pltpu
from jax.experimental.pallas import tpu_sc as plsc

K = 1000000
D = 16
NTILES = 32
CHUNK = 2048
SC_ROWS = 999936
NFULL = SC_ROWS // CHUNK
HALF_START = NFULL * CHUNK
HALF_ROWS = SC_ROWS - HALF_START
HALF_TILE = NFULL % NTILES
TC_START = SC_ROWS


def _iota16():
    return lax.iota(jnp.int32, 16)


def _lane_gather(v, idx16):
    dn = lax.GatherDimensionNumbers(
        offset_dims=(), collapsed_slice_dims=(0,), start_index_map=(0,))
    return lax.gather(v, idx16[:, None], dn, slice_sizes=(1,),
                      mode=lax.GatherScatterMode.PROMISE_IN_BOUNDS)


def _splat(v, lane):
    return _lane_gather(v, jnp.full((16,), lane, jnp.int32))


def _merge_topk(cur_v, cur_i, blk_v, blk_i):
    sk, si = plsc.sort_key_val(blk_v, blk_i)
    rv = lax.rev(sk, (0,))
    ri = lax.rev(si, (0,))
    take_a = cur_v <= rv
    mv = jnp.where(take_a, cur_v, rv)
    mi = jnp.where(take_a, cur_i, ri)
    nv, ni = plsc.sort_key_val(mv, mi)
    return nv, ni, _splat(nv, 7)


def _sc_body(mnvT_hbm, smp_hbm, outv_hbm, outi_hbm,
             bufa, bufb, sbuf, resv, resi, sema, semb):
    cid = lax.axis_index("c")
    sid = lax.axis_index("s")
    wid = sid * 2 + cid

    pltpu.sync_copy(smp_hbm, sbuf)
    s = sbuf[...]
    iota = _iota16()
    inf = jnp.float32(jnp.inf)
    s_d = [_splat(s, d) for d in range(D)]

    def block_dist2(buf, b):
        lr0 = b * 16
        accs = [jnp.zeros((16,), jnp.float32) for _ in range(4)]
        for d in range(D):
            col = buf[d, pl.ds(lr0, 16)]
            diff = col - s_d[d]
            accs[d & 3] = accs[d & 3] + diff * diff
        return (accs[0] + accs[1]) + (accs[2] + accs[3])

    def make_blk_body(buf, ok, row_start):
        def blk_body(b4, carry):
            cur_v, cur_i, tv = carry
            b = b4 * 4
            accs = [block_dist2(buf, b + q) for q in range(4)]
            gidx0 = row_start + b * 16 + iota
            lo = jnp.minimum(jnp.minimum(accs[0], accs[1]),
                             jnp.minimum(accs[2], accs[3]))
            hit = ok & jnp.any(lo < tv)

            def do(a0, a1, a2, a3, gidx0, cur_v, cur_i, tv):
                for q, a in enumerate((a0, a1, a2, a3)):
                    cur_v, cur_i, tv = _merge_topk(
                        cur_v, cur_i, a, gidx0 + q * 16)
                return cur_v, cur_i, tv

            def skip(a0, a1, a2, a3, gidx0, cur_v, cur_i, tv):
                return cur_v, cur_i, tv

            return lax.cond(hit, do, skip,
                            *accs, gidx0, cur_v, cur_i, tv)
        return blk_body

    def slot_chunk(j):
        ch = wid + j * NTILES
        ok = ch < NFULL
        return jnp.where(ok, ch, wid), ok

    def dma(ch, buf, sem):
        return pltpu.make_async_copy(
            mnvT_hbm.at[:, pl.ds(ch * CHUNK, CHUNK)], buf, sem)

    def compute(ch, ok, buf, carry):
        return lax.fori_loop(
            0, CHUNK // 64, make_blk_body(buf, ok, ch * CHUNK), carry)

    NSLOT = (NFULL + NTILES - 1) // NTILES

    ch0, _ = slot_chunk(0)
    dma(ch0, bufa, sema).start()

    def pair_body(p, carry):
        j0 = p * 2
        ch0, ok0 = slot_chunk(j0)
        ch1, ok1 = slot_chunk(j0 + 1)
        ch2, _ = slot_chunk(j0 + 2)
        dma(ch0, bufa, sema).wait()
        dma(ch1, bufb, semb).start()
        carry = compute(ch0, ok0, bufa, carry)
        dma(ch1, bufb, semb).wait()
        dma(ch2, bufa, sema).start()
        return compute(ch1, ok1, bufb, carry)

    init = (jnp.full((16,), inf), jnp.zeros((16,), jnp.int32),
            jnp.full((16,), inf))
    cur_v, cur_i, tv = lax.fori_loop(0, NSLOT // 2, pair_body, init)
    chx, _ = slot_chunk(NSLOT)
    dma(chx, bufa, sema).wait()

    @pl.when(wid == HALF_TILE)
    def _():
        pltpu.sync_copy(
            mnvT_hbm.at[:, pl.ds(HALF_START, HALF_ROWS)],
            bufa.at[:, pl.ds(0, HALF_ROWS)])
        cv, ci, _t = lax.fori_loop(
            0, HALF_ROWS // 64,
            make_blk_body(bufa, True, HALF_START), (cur_v, cur_i, tv))
        resv[...] = cv
        resi[...] = ci

    @pl.when(wid != HALF_TILE)
    def _():
        resv[...] = cur_v
        resi[...] = cur_i

    pltpu.sync_copy(resv, outv_hbm.at[pl.ds(wid * 16, 16)])
    pltpu.sync_copy(resi, outi_hbm.at[pl.ds(wid * 16, 16)])


def _sc_dist_topk(mnvT, smp):
    mesh = plsc.VectorSubcoreMesh(core_axis_name="c", subcore_axis_name="s")
    f = pl.kernel(
        _sc_body,
        mesh=mesh,
        compiler_params=pltpu.CompilerParams(
            needs_layout_passes=False, use_tc_tiling_on_sc=True),
        out_type=[
            jax.ShapeDtypeStruct((NTILES * 16,), jnp.float32),
            jax.ShapeDtypeStruct((NTILES * 16,), jnp.int32),
        ],
        scratch_types=[
            pltpu.VMEM((D, CHUNK), jnp.float32),
            pltpu.VMEM((D, CHUNK), jnp.float32),
            pltpu.VMEM((16,), jnp.float32),
            pltpu.VMEM((16,), jnp.float32),
            pltpu.VMEM((16,), jnp.int32),
            pltpu.SemaphoreType.DMA,
            pltpu.SemaphoreType.DMA,
        ],
    )
    return f(mnvT, smp)


def _tc_merge_body(v_ref, i_ref, tail_ref, s_ref, idx_ref, val_ref):
    t = tail_ref[...]
    sv = s_ref[...]
    td = jnp.sum((t - sv) ** 2, axis=1)
    td = td.reshape(4, 16)
    ti = TC_START + lax.broadcasted_iota(jnp.int32, (4, 16), 0) * 16 \
        + lax.broadcasted_iota(jnp.int32, (4, 16), 1)
    V = jnp.concatenate([v_ref[...], td], axis=0)
    I = jnp.concatenate([i_ref[...], ti], axis=0)
    big = jnp.int32(2**31 - 1)
    inf = jnp.float32(jnp.inf)
    idxs = []
    vals = []
    for _ in range(8):
        m = jnp.min(V)
        sel = V == m
        ci = jnp.min(jnp.where(sel, I, big))
        idxs.append(ci)
        vals.append(m)
        V = jnp.where(sel & (I == ci), inf, V)
    idx_ref[...] = jnp.stack(idxs)
    val_ref[...] = jnp.sqrt(jnp.stack(vals))


def _tc_merge(v2d, i2d, tail, smp):
    return pl.pallas_call(
        _tc_merge_body,
        out_shape=[
            jax.ShapeDtypeStruct((8,), jnp.int32),
            jax.ShapeDtypeStruct((8,), jnp.float32),
        ],
    )(v2d, i2d, tail, smp)


def kernel(samples, map_node_values, n):
    mnvT = map_node_values.T
    cv, ci = _sc_dist_topk(mnvT, samples)
    tail = map_node_values[TC_START:]
    idx, vals = _tc_merge(cv.reshape(NTILES, 16), ci.reshape(NTILES, 16),
                          tail, samples.reshape(1, D))
    return idx, vals
